# baseline (speedup 1.0000x reference)
import functools

import jax
import jax.numpy as jnp
from jax import lax
from jax.experimental import pallas as pl
from jax.experimental.pallas import tpu as pltpu
from jax.experimental.pallas import tpu_sc as plsc

KNN = 16
EPS = 1e-5

SC_CORES = 2
SC_SUBCORES = 16
SC_WORKERS = SC_CORES * SC_SUBCORES


def _q_body(x_ref, w_ref, o_ref):
    o_ref[...] = jnp.dot(x_ref[...], w_ref[...],
                         preferred_element_type=jnp.float32)


def _topk_body(xyz_ref, xyzt_ref, o_ref):
    b = pl.program_id(0)
    x = xyz_ref[0]
    xt = xyzt_ref[0]
    n = xt.shape[1]
    sq_t = jnp.sum(x * x, axis=1, keepdims=True)
    sq_f = jnp.sum(xt * xt, axis=0, keepdims=True)
    dot = jnp.dot(x, xt, preferred_element_type=jnp.float32)
    d2 = jnp.maximum(sq_t + sq_f - 2.0 * dot, 0.0)
    iota = lax.broadcasted_iota(jnp.int32, d2.shape, 1)
    cols = []
    for _ in range(KNN):
        m = jnp.min(d2, axis=1, keepdims=True)
        cand = jnp.where(d2 == m, iota, n)
        i = jnp.min(cand, axis=1, keepdims=True)
        cols.append(i)
        d2 = jnp.where(cand == i, jnp.float32(jnp.inf), d2)
    idx = jnp.concatenate(cols, axis=1)
    o_ref[0] = idx + b * n


def _sc_gather(qflat, idxflat):
    rows, d = idxflat.shape[0], qflat.shape[1]
    ch = 128
    chunks = rows // (SC_WORKERS * ch)

    mesh = plsc.VectorSubcoreMesh(core_axis_name="c", subcore_axis_name="s")

    @functools.partial(
        pl.kernel,
        out_type=jax.ShapeDtypeStruct((rows, d), qflat.dtype),
        mesh=mesh,
        scratch_types=[
            pltpu.VMEM((ch,), jnp.int32),
            pltpu.VMEM((ch, d), qflat.dtype),
            pltpu.SemaphoreType.DMA,
        ],
    )
    def gk(q_hbm, idx_hbm, out_hbm, idx_v, rows_v, sem):
        c = lax.axis_index("c")
        s = lax.axis_index("s")
        wid = s * SC_CORES + c

        def body(j, carry):
            base = (wid * chunks + j) * ch
            pltpu.sync_copy(idx_hbm.at[pl.ds(base, ch)], idx_v)
            pltpu.async_copy(q_hbm.at[idx_v], rows_v, sem).wait()
            pltpu.sync_copy(rows_v, out_hbm.at[pl.ds(base, ch)])
            return carry

        lax.fori_loop(0, chunks, body, 0)

    return gk(qflat, idxflat)


def _main_body(nrows, g_ref, q_ref, b1_ref, g1_ref, be1_ref, w2_ref, b2_ref,
               mmax_ref, mmin_ref, st2_ref, acc_ref):
    p = pl.program_id(0)
    t = pl.program_id(1)
    last_t = pl.num_programs(1) - 1
    tn = q_ref.shape[0]
    d = q_ref.shape[1]

    g = g_ref[...].astype(jnp.float32)
    qb = jnp.broadcast_to(q_ref[...][:, None, :],
                          (tn, KNN, d)).reshape(tn * KNN, d)
    h1 = g - qb + b1_ref[...]

    @pl.when(jnp.logical_and(p == 0, t == 0))
    def _():
        acc_ref[...] = jnp.zeros_like(acc_ref)

    @pl.when(p == 0)
    def _():
        s = jnp.sum(h1, axis=0, keepdims=True)
        ss = jnp.sum(h1 * h1, axis=0, keepdims=True)
        acc_ref[0:2, :] += jnp.concatenate([s, ss], axis=0)

    @pl.when(p == 1)
    def _():
        st = acc_ref[0:2, :]
        mean1 = st[0:1, :] / nrows
        var1 = st[1:2, :] / nrows - mean1 * mean1
        sc1 = g1_ref[...] * lax.rsqrt(var1 + EPS)
        c1 = be1_ref[...] - mean1 * sc1
        a = jnp.maximum(h1 * sc1 + c1, 0.0).astype(jnp.bfloat16)
        h2 = jnp.dot(a, w2_ref[...],
                     preferred_element_type=jnp.float32) + b2_ref[...]
        s = jnp.sum(h2, axis=0, keepdims=True)
        ss = jnp.sum(h2 * h2, axis=0, keepdims=True)
        acc_ref[2:4, :] += jnp.concatenate([s, ss], axis=0)
        h3 = h2.reshape(tn, KNN, d)
        mmax_ref[...] = jnp.max(h3, axis=1)
        mmin_ref[...] = jnp.min(h3, axis=1)

        @pl.when(t == last_t)
        def _():
            st2_ref[...] = acc_ref[2:4, :]


def _final_body(nrows, mmax_ref, mmin_ref, st2_ref, g2_ref, be2_ref, o_ref):
    st = st2_ref[...]
    mean2 = st[0:1, :] / nrows
    var2 = st[1:2, :] / nrows - mean2 * mean2
    g2 = g2_ref[...]
    sc2 = g2 * lax.rsqrt(var2 + EPS)
    c2 = be2_ref[...] - mean2 * sc2
    pick = jnp.where(g2 >= 0.0, mmax_ref[...], mmin_ref[...])
    o_ref[...] = jnp.maximum(pick * sc2 + c2, 0.0)


def kernel(xyz, features, W1, b1, g1, be1, W2, b2, g2, be2):
    B, N, _ = xyz.shape
    D = W1.shape[1]
    R = B * N * KNN
    BN = B * N

    x_cat = jnp.concatenate([xyz, features], axis=2).reshape(BN, D)
    b1r = b1.reshape(1, D)
    g1r = g1.reshape(1, D)
    be1r = be1.reshape(1, D)
    b2r = b2.reshape(1, D)
    g2r = g2.reshape(1, D)
    be2r = be2.reshape(1, D)
    w2bf = W2.astype(jnp.bfloat16)

    TQ = 1024
    q = pl.pallas_call(
        _q_body,
        grid=(BN // TQ,),
        in_specs=[
            pl.BlockSpec((TQ, D), lambda t: (t, 0)),
            pl.BlockSpec((D, D), lambda t: (0, 0)),
        ],
        out_specs=pl.BlockSpec((TQ, D), lambda t: (t, 0)),
        out_shape=jax.ShapeDtypeStruct((BN, D), jnp.float32),
    )(x_cat, W1)

    TN = 256
    xyzt = xyz.transpose(0, 2, 1)
    idx = pl.pallas_call(
        _topk_body,
        grid=(B, N // TN),
        in_specs=[
            pl.BlockSpec((1, TN, 3), lambda b, t: (b, t, 0)),
            pl.BlockSpec((1, 3, N), lambda b, t: (b, 0, 0)),
        ],
        out_specs=pl.BlockSpec((1, TN, KNN), lambda b, t: (b, t, 0)),
        out_shape=jax.ShapeDtypeStruct((B, N, KNN), jnp.int32),
    )(xyz, xyzt)

    g_rows = _sc_gather(q, idx.reshape(R))

    TM = 128
    mmax, mmin, stats2 = pl.pallas_call(
        functools.partial(_main_body, float(R)),
        grid=(2, BN // TM),
        in_specs=[
            pl.BlockSpec((TM * KNN, D), lambda p, t: (t, 0)),
            pl.BlockSpec((TM, D), lambda p, t: (t, 0)),
            pl.BlockSpec((1, D), lambda p, t: (0, 0)),
            pl.BlockSpec((1, D), lambda p, t: (0, 0)),
            pl.BlockSpec((1, D), lambda p, t: (0, 0)),
            pl.BlockSpec((D, D), lambda p, t: (0, 0)),
            pl.BlockSpec((1, D), lambda p, t: (0, 0)),
        ],
        out_specs=[
            pl.BlockSpec((TM, D), lambda p, t: (t, 0)),
            pl.BlockSpec((TM, D), lambda p, t: (t, 0)),
            pl.BlockSpec((2, D), lambda p, t: (0, 0)),
        ],
        out_shape=[
            jax.ShapeDtypeStruct((BN, D), jnp.float32),
            jax.ShapeDtypeStruct((BN, D), jnp.float32),
            jax.ShapeDtypeStruct((2, D), jnp.float32),
        ],
        scratch_shapes=[pltpu.VMEM((4, D), jnp.float32)],
    )(g_rows, q, b1r, g1r, be1r, w2bf, b2r)

    TF = 512
    out = pl.pallas_call(
        functools.partial(_final_body, float(R)),
        grid=(BN // TF,),
        in_specs=[
            pl.BlockSpec((TF, D), lambda t: (t, 0)),
            pl.BlockSpec((TF, D), lambda t: (t, 0)),
            pl.BlockSpec((2, D), lambda t: (0, 0)),
            pl.BlockSpec((1, D), lambda t: (0, 0)),
            pl.BlockSpec((1, D), lambda t: (0, 0)),
        ],
        out_specs=pl.BlockSpec((TF, D), lambda t: (t, 0)),
        out_shape=jax.ShapeDtypeStruct((BN, D), jnp.float32),
    )(mmax, mmin, stats2, g2r, be2r)

    return out.reshape(B, N, D)


# speedup vs baseline: 1.0496x; 1.0496x over previous
import functools

import jax
import jax.numpy as jnp
from jax import lax
from jax.experimental import pallas as pl
from jax.experimental.pallas import tpu as pltpu
from jax.experimental.pallas import tpu_sc as plsc

KNN = 16
EPS = 1e-5

SC_CORES = 2
SC_SUBCORES = 16
SC_WORKERS = SC_CORES * SC_SUBCORES


def _q_body(x_ref, w_ref, o_ref):
    o_ref[...] = jnp.dot(x_ref[...], w_ref[...],
                         preferred_element_type=jnp.float32)


def _topk_body(xyz_ref, xyzt_ref, o_ref):
    b = pl.program_id(0)
    x = xyz_ref[0]
    xt = xyzt_ref[0]
    n = xt.shape[1]
    sq_t = jnp.sum(x * x, axis=1, keepdims=True)
    sq_f = jnp.sum(xt * xt, axis=0, keepdims=True)
    dot = jnp.dot(x, xt, preferred_element_type=jnp.float32)
    d2 = jnp.maximum(sq_t + sq_f - 2.0 * dot, 0.0)
    iota = lax.broadcasted_iota(jnp.int32, d2.shape, 1)
    cols = []
    for _ in range(KNN):
        m = jnp.min(d2, axis=1, keepdims=True)
        cand = jnp.where(d2 == m, iota, n)
        i = jnp.min(cand, axis=1, keepdims=True)
        cols.append(i)
        d2 = jnp.where(cand == i, jnp.float32(jnp.inf), d2)
    idx = jnp.concatenate(cols, axis=1)
    o_ref[0] = idx + b * n


def _sc_gather(qflat, idxflat):
    rows, d = idxflat.shape[0], qflat.shape[1]
    ch = 128
    chunks = rows // (SC_WORKERS * ch)

    mesh = plsc.VectorSubcoreMesh(core_axis_name="c", subcore_axis_name="s")

    @functools.partial(
        pl.kernel,
        out_type=jax.ShapeDtypeStruct((rows, d), jnp.float32),
        mesh=mesh,
        scratch_types=[
            pltpu.VMEM((ch,), jnp.int32),
            pltpu.VMEM((ch, d), jnp.float32),
            pltpu.SemaphoreType.DMA,
        ],
    )
    def gk(q_hbm, idx_hbm, out_hbm, idx_v, rows_v, sem):
        c = lax.axis_index("c")
        s = lax.axis_index("s")
        wid = s * SC_CORES + c

        def body(j, carry):
            base = (wid * chunks + j) * ch
            pltpu.sync_copy(idx_hbm.at[pl.ds(base, ch)], idx_v)
            pltpu.async_copy(q_hbm.at[idx_v], rows_v, sem).wait()
            pltpu.sync_copy(rows_v, out_hbm.at[pl.ds(base, ch)])
            return carry

        lax.fori_loop(0, chunks, body, 0)

    return gk(qflat, idxflat)


def _stats1_body(g_ref, q_ref, b1_ref, o_ref):
    t = pl.program_id(0)
    tn = q_ref.shape[0]
    d = q_ref.shape[1]
    g = g_ref[...]
    qb = jnp.broadcast_to(q_ref[...][:, None, :],
                          (tn, KNN, d)).reshape(tn * KNN, d)
    h1 = g - qb + b1_ref[...]
    s = jnp.sum(h1, axis=0, keepdims=True)
    ss = jnp.sum(h1 * h1, axis=0, keepdims=True)
    upd = jnp.concatenate([s, ss], axis=0)

    @pl.when(t == 0)
    def _():
        o_ref[...] = jnp.zeros_like(o_ref)

    o_ref[...] += upd


def _main_body(g_ref, q_ref, st1_ref, b1_ref, g1_ref, be1_ref, w2_ref,
               b2_ref, nrows_ref, mmax_ref, mmin_ref, st2_ref):
    t = pl.program_id(0)
    tn = q_ref.shape[0]
    d = q_ref.shape[1]
    nrows = nrows_ref[0, 0]
    st = st1_ref[...]
    mean1 = st[0:1, :] / nrows
    var1 = st[1:2, :] / nrows - mean1 * mean1
    sc1 = g1_ref[...] * lax.rsqrt(var1 + EPS)
    c1 = be1_ref[...] - mean1 * sc1

    g = g_ref[...]
    qb = jnp.broadcast_to(q_ref[...][:, None, :],
                          (tn, KNN, d)).reshape(tn * KNN, d)
    h1 = g - qb + b1_ref[...]
    a = jnp.maximum(h1 * sc1 + c1, 0.0).astype(jnp.bfloat16)
    h2 = jnp.dot(a, w2_ref[...],
                 preferred_element_type=jnp.float32) + b2_ref[...]

    s = jnp.sum(h2, axis=0, keepdims=True)
    ss = jnp.sum(h2 * h2, axis=0, keepdims=True)

    @pl.when(t == 0)
    def _():
        st2_ref[...] = jnp.zeros_like(st2_ref)

    st2_ref[...] += jnp.concatenate([s, ss], axis=0)

    h3 = h2.reshape(tn, KNN, d)
    mmax_ref[...] = jnp.max(h3, axis=1)
    mmin_ref[...] = jnp.min(h3, axis=1)


def _final_body(mmax_ref, mmin_ref, st2_ref, g2_ref, be2_ref, nrows_ref,
                o_ref):
    nrows = nrows_ref[0, 0]
    st = st2_ref[...]
    mean2 = st[0:1, :] / nrows
    var2 = st[1:2, :] / nrows - mean2 * mean2
    g2 = g2_ref[...]
    sc2 = g2 * lax.rsqrt(var2 + EPS)
    c2 = be2_ref[...] - mean2 * sc2
    pick = jnp.where(g2 >= 0.0, mmax_ref[...], mmin_ref[...])
    o_ref[...] = jnp.maximum(pick * sc2 + c2, 0.0)


def kernel(xyz, features, W1, b1, g1, be1, W2, b2, g2, be2):
    B, N, _ = xyz.shape
    D = W1.shape[1]
    R = B * N * KNN
    BN = B * N

    x_cat = jnp.concatenate([xyz, features], axis=2).reshape(BN, D)
    b1r = b1.reshape(1, D)
    g1r = g1.reshape(1, D)
    be1r = be1.reshape(1, D)
    b2r = b2.reshape(1, D)
    g2r = g2.reshape(1, D)
    be2r = be2.reshape(1, D)
    nrows = jnp.full((1, 1), float(R), dtype=jnp.float32)
    w2bf = W2.astype(jnp.bfloat16)

    TQ = 1024
    q = pl.pallas_call(
        _q_body,
        grid=(BN // TQ,),
        in_specs=[
            pl.BlockSpec((TQ, D), lambda t: (t, 0)),
            pl.BlockSpec((D, D), lambda t: (0, 0)),
        ],
        out_specs=pl.BlockSpec((TQ, D), lambda t: (t, 0)),
        out_shape=jax.ShapeDtypeStruct((BN, D), jnp.float32),
    )(x_cat, W1)

    TN = 256
    xyzt = xyz.transpose(0, 2, 1)
    idx = pl.pallas_call(
        _topk_body,
        grid=(B, N // TN),
        in_specs=[
            pl.BlockSpec((1, TN, 3), lambda b, t: (b, t, 0)),
            pl.BlockSpec((1, 3, N), lambda b, t: (b, 0, 0)),
        ],
        out_specs=pl.BlockSpec((1, TN, KNN), lambda b, t: (b, t, 0)),
        out_shape=jax.ShapeDtypeStruct((B, N, KNN), jnp.int32),
    )(xyz, xyzt)

    g_rows = _sc_gather(q, idx.reshape(R))

    TS = 128
    stats1 = pl.pallas_call(
        _stats1_body,
        grid=(BN // TS,),
        in_specs=[
            pl.BlockSpec((TS * KNN, D), lambda t: (t, 0)),
            pl.BlockSpec((TS, D), lambda t: (t, 0)),
            pl.BlockSpec((1, D), lambda t: (0, 0)),
        ],
        out_specs=pl.BlockSpec((2, D), lambda t: (0, 0)),
        out_shape=jax.ShapeDtypeStruct((2, D), jnp.float32),
    )(g_rows, q, b1r)

    TM = 128
    mmax, mmin, stats2 = pl.pallas_call(
        _main_body,
        grid=(BN // TM,),
        in_specs=[
            pl.BlockSpec((TM * KNN, D), lambda t: (t, 0)),
            pl.BlockSpec((TM, D), lambda t: (t, 0)),
            pl.BlockSpec((2, D), lambda t: (0, 0)),
            pl.BlockSpec((1, D), lambda t: (0, 0)),
            pl.BlockSpec((1, D), lambda t: (0, 0)),
            pl.BlockSpec((1, D), lambda t: (0, 0)),
            pl.BlockSpec((D, D), lambda t: (0, 0)),
            pl.BlockSpec((1, D), lambda t: (0, 0)),
            pl.BlockSpec((1, 1), lambda t: (0, 0), memory_space=pltpu.SMEM),
        ],
        out_specs=[
            pl.BlockSpec((TM, D), lambda t: (t, 0)),
            pl.BlockSpec((TM, D), lambda t: (t, 0)),
            pl.BlockSpec((2, D), lambda t: (0, 0)),
        ],
        out_shape=[
            jax.ShapeDtypeStruct((BN, D), jnp.float32),
            jax.ShapeDtypeStruct((BN, D), jnp.float32),
            jax.ShapeDtypeStruct((2, D), jnp.float32),
        ],
    )(g_rows, q, stats1, b1r, g1r, be1r, w2bf, b2r, nrows)

    TF = 512
    out = pl.pallas_call(
        _final_body,
        grid=(BN // TF,),
        in_specs=[
            pl.BlockSpec((TF, D), lambda t: (t, 0)),
            pl.BlockSpec((TF, D), lambda t: (t, 0)),
            pl.BlockSpec((2, D), lambda t: (0, 0)),
            pl.BlockSpec((1, D), lambda t: (0, 0)),
            pl.BlockSpec((1, D), lambda t: (0, 0)),
            pl.BlockSpec((1, 1), lambda t: (0, 0), memory_space=pltpu.SMEM),
        ],
        out_specs=pl.BlockSpec((TF, D), lambda t: (t, 0)),
        out_shape=jax.ShapeDtypeStruct((BN, D), jnp.float32),
    )(mmax, mmin, stats2, g2r, be2r, nrows)

    return out.reshape(B, N, D)


# speedup vs baseline: 1.1739x; 1.1184x over previous
import functools

import jax
import jax.numpy as jnp
from jax import lax
from jax.experimental import pallas as pl
from jax.experimental.pallas import tpu as pltpu
from jax.experimental.pallas import tpu_sc as plsc

KNN = 16
EPS = 1e-5

SC_CORES = 2
SC_SUBCORES = 16
SC_WORKERS = SC_CORES * SC_SUBCORES


def _q_body(x_ref, w_ref, o_ref):
    o_ref[...] = jnp.dot(x_ref[...], w_ref[...],
                         preferred_element_type=jnp.float32)


def _topk_body(xyz_ref, xyzt_ref, o_ref):
    x = xyz_ref[...]
    xt = xyzt_ref[...]
    n = xt.shape[1]
    sq_t = jnp.sum(x * x, axis=1, keepdims=True)
    sq_f = jnp.sum(xt * xt, axis=0, keepdims=True)
    dot = jnp.dot(x, xt, preferred_element_type=jnp.float32)
    d2 = jnp.maximum(sq_t + sq_f - 2.0 * dot, 0.0)
    iota = lax.broadcasted_iota(jnp.int32, d2.shape, 1)
    cols = []
    for _ in range(KNN):
        m = jnp.min(d2, axis=1, keepdims=True)
        cand = jnp.where(d2 == m, iota, n)
        i = jnp.min(cand, axis=1, keepdims=True)
        cols.append(i)
        d2 = jnp.where(cand == i, jnp.float32(jnp.inf), d2)
    o_ref[...] = jnp.concatenate(cols, axis=1)


def _sc_gather(qflat, idxflat):
    rows, d = idxflat.shape[0], qflat.shape[1]
    ch = 128
    chunks = rows // (SC_WORKERS * ch)

    mesh = plsc.VectorSubcoreMesh(core_axis_name="c", subcore_axis_name="s")

    @functools.partial(
        pl.kernel,
        out_type=jax.ShapeDtypeStruct((rows, d), qflat.dtype),
        mesh=mesh,
        scratch_types=[
            pltpu.VMEM((ch,), jnp.int32),
            pltpu.VMEM((ch, d), qflat.dtype),
            pltpu.SemaphoreType.DMA,
        ],
    )
    def gk(q_hbm, idx_hbm, out_hbm, idx_v, rows_v, sem):
        c = lax.axis_index("c")
        s = lax.axis_index("s")
        wid = s * SC_CORES + c

        def body(j, carry):
            base = (wid * chunks + j) * ch
            pltpu.sync_copy(idx_hbm.at[pl.ds(base, ch)], idx_v)
            pltpu.async_copy(q_hbm.at[idx_v], rows_v, sem).wait()
            pltpu.sync_copy(rows_v, out_hbm.at[pl.ds(base, ch)])
            return carry

        lax.fori_loop(0, chunks, body, 0)

    return gk(qflat, idxflat)


def _stats1_body(g_ref, q_ref, b1_ref, o_ref):
    t = pl.program_id(0)
    tn = q_ref.shape[0]
    d = q_ref.shape[1]
    g = g_ref[...]
    qb = jnp.broadcast_to(q_ref[...][:, None, :],
                          (tn, KNN, d)).reshape(tn * KNN, d)
    h1 = g - qb + b1_ref[...]
    s = jnp.sum(h1, axis=0, keepdims=True)
    ss = jnp.sum(h1 * h1, axis=0, keepdims=True)
    upd = jnp.concatenate([s, ss], axis=0)

    @pl.when(t == 0)
    def _():
        o_ref[...] = jnp.zeros_like(o_ref)

    o_ref[...] += upd


def _main_body(nrows, g_ref, q_ref, st1_ref, b1_ref, g1_ref, be1_ref, w2_ref,
               b2_ref, mmax_ref, mmin_ref, st2_ref):
    t = pl.program_id(0)
    tn = q_ref.shape[0]
    d = q_ref.shape[1]
    st = st1_ref[...]
    mean1 = st[0:1, :] / nrows
    var1 = st[1:2, :] / nrows - mean1 * mean1
    sc1 = g1_ref[...] * lax.rsqrt(var1 + EPS)
    c1 = be1_ref[...] - mean1 * sc1

    g = g_ref[...]
    qb = jnp.broadcast_to(q_ref[...][:, None, :],
                          (tn, KNN, d)).reshape(tn * KNN, d)
    h1 = g - qb + b1_ref[...]
    a = jnp.maximum(h1 * sc1 + c1, 0.0).astype(jnp.bfloat16)
    h2 = jnp.dot(a, w2_ref[...],
                 preferred_element_type=jnp.float32) + b2_ref[...]

    s = jnp.sum(h2, axis=0, keepdims=True)
    ss = jnp.sum(h2 * h2, axis=0, keepdims=True)

    @pl.when(t == 0)
    def _():
        st2_ref[...] = jnp.zeros_like(st2_ref)

    st2_ref[...] += jnp.concatenate([s, ss], axis=0)

    h3 = h2.reshape(tn, KNN, d)
    mmax_ref[...] = jnp.max(h3, axis=1)
    mmin_ref[...] = jnp.min(h3, axis=1)


def _final_body(nrows, mmax_ref, mmin_ref, st2_ref, g2_ref, be2_ref, o_ref):
    st = st2_ref[...]
    mean2 = st[0:1, :] / nrows
    var2 = st[1:2, :] / nrows - mean2 * mean2
    g2 = g2_ref[...]
    sc2 = g2 * lax.rsqrt(var2 + EPS)
    c2 = be2_ref[...] - mean2 * sc2
    pick = jnp.where(g2 >= 0.0, mmax_ref[...], mmin_ref[...])
    o_ref[...] = jnp.maximum(pick * sc2 + c2, 0.0)


def kernel(xyz, features, W1, b1, g1, be1, W2, b2, g2, be2):
    B, N, _ = xyz.shape
    D = W1.shape[1]
    R = B * N * KNN
    BN = B * N
    RB = N * KNN

    x_cat = jnp.concatenate([xyz, features], axis=2).reshape(BN, D)
    b1r = b1.reshape(1, D)
    g1r = g1.reshape(1, D)
    be1r = be1.reshape(1, D)
    b2r = b2.reshape(1, D)
    g2r = g2.reshape(1, D)
    be2r = be2.reshape(1, D)
    w2bf = W2.astype(jnp.bfloat16)

    TQ = 1024
    q = pl.pallas_call(
        _q_body,
        grid=(BN // TQ,),
        in_specs=[
            pl.BlockSpec((TQ, D), lambda t: (t, 0)),
            pl.BlockSpec((D, D), lambda t: (0, 0)),
        ],
        out_specs=pl.BlockSpec((TQ, D), lambda t: (t, 0)),
        out_shape=jax.ShapeDtypeStruct((BN, D), jnp.float32),
    )(x_cat, W1)
    q = q.reshape(B, N, D)

    TN = 256
    xyzt = xyz.transpose(0, 2, 1)
    topk_call = pl.pallas_call(
        _topk_body,
        grid=(N // TN,),
        in_specs=[
            pl.BlockSpec((TN, 3), lambda t: (t, 0)),
            pl.BlockSpec((3, N), lambda t: (0, 0)),
        ],
        out_specs=pl.BlockSpec((TN, KNN), lambda t: (t, 0)),
        out_shape=jax.ShapeDtypeStruct((N, KNN), jnp.int32),
    )
    g_rows = []
    for b in range(B):
        idx_b = topk_call(xyz[b], xyzt[b])
        g_rows.append(_sc_gather(q[b], idx_b.reshape(RB)))

    TS = 128
    stats_call = pl.pallas_call(
        _stats1_body,
        grid=(N // TS,),
        in_specs=[
            pl.BlockSpec((TS * KNN, D), lambda t: (t, 0)),
            pl.BlockSpec((TS, D), lambda t: (t, 0)),
            pl.BlockSpec((1, D), lambda t: (0, 0)),
        ],
        out_specs=pl.BlockSpec((2, D), lambda t: (0, 0)),
        out_shape=jax.ShapeDtypeStruct((2, D), jnp.float32),
    )
    stats1 = sum(stats_call(g_rows[b], q[b], b1r) for b in range(B))

    TM = 128
    main_call = pl.pallas_call(
        functools.partial(_main_body, float(R)),
        grid=(N // TM,),
        in_specs=[
            pl.BlockSpec((TM * KNN, D), lambda t: (t, 0)),
            pl.BlockSpec((TM, D), lambda t: (t, 0)),
            pl.BlockSpec((2, D), lambda t: (0, 0)),
            pl.BlockSpec((1, D), lambda t: (0, 0)),
            pl.BlockSpec((1, D), lambda t: (0, 0)),
            pl.BlockSpec((1, D), lambda t: (0, 0)),
            pl.BlockSpec((D, D), lambda t: (0, 0)),
            pl.BlockSpec((1, D), lambda t: (0, 0)),
        ],
        out_specs=[
            pl.BlockSpec((TM, D), lambda t: (t, 0)),
            pl.BlockSpec((TM, D), lambda t: (t, 0)),
            pl.BlockSpec((2, D), lambda t: (0, 0)),
        ],
        out_shape=[
            jax.ShapeDtypeStruct((N, D), jnp.float32),
            jax.ShapeDtypeStruct((N, D), jnp.float32),
            jax.ShapeDtypeStruct((2, D), jnp.float32),
        ],
    )
    mmax, mmin, st2 = [], [], []
    for b in range(B):
        a_, b_, c_ = main_call(g_rows[b], q[b], stats1, b1r, g1r, be1r,
                               w2bf, b2r)
        mmax.append(a_)
        mmin.append(b_)
        st2.append(c_)
    stats2 = sum(st2)

    TF = 512
    final_call = pl.pallas_call(
        functools.partial(_final_body, float(R)),
        grid=(N // TF,),
        in_specs=[
            pl.BlockSpec((TF, D), lambda t: (t, 0)),
            pl.BlockSpec((TF, D), lambda t: (t, 0)),
            pl.BlockSpec((2, D), lambda t: (0, 0)),
            pl.BlockSpec((1, D), lambda t: (0, 0)),
            pl.BlockSpec((1, D), lambda t: (0, 0)),
        ],
        out_specs=pl.BlockSpec((TF, D), lambda t: (t, 0)),
        out_shape=jax.ShapeDtypeStruct((N, D), jnp.float32),
    )
    out = [final_call(mmax[b], mmin[b], stats2, g2r, be2r) for b in range(B)]
    return jnp.stack(out, axis=0)


# speedup vs baseline: 1.2350x; 1.0520x over previous
import functools

import jax
import jax.numpy as jnp
from jax import lax
from jax.experimental import pallas as pl
from jax.experimental.pallas import tpu as pltpu
from jax.experimental.pallas import tpu_sc as plsc

KNN = 16
EPS = 1e-5

SC_CORES = 2
SC_SUBCORES = 16
SC_WORKERS = SC_CORES * SC_SUBCORES


def _q_body(x_ref, w_ref, o_ref):
    o_ref[...] = jnp.dot(x_ref[...], w_ref[...],
                         preferred_element_type=jnp.float32)


def _topk_body(xyz_ref, xyzt_ref, o_ref):
    x = xyz_ref[...]
    xt = xyzt_ref[...]
    n = xt.shape[1]
    sq_t = jnp.sum(x * x, axis=1, keepdims=True)
    sq_f = jnp.sum(xt * xt, axis=0, keepdims=True)
    dot = jnp.dot(x, xt, preferred_element_type=jnp.float32)
    d2 = jnp.maximum(sq_t + sq_f - 2.0 * dot, 0.0)
    iota = lax.broadcasted_iota(jnp.int32, d2.shape, 1)
    cols = []
    for _ in range(KNN):
        m = jnp.min(d2, axis=1, keepdims=True)
        cand = jnp.where(d2 == m, iota, n)
        i = jnp.min(cand, axis=1, keepdims=True)
        cols.append(i)
        d2 = jnp.where(cand == i, jnp.float32(jnp.inf), d2)
    o_ref[...] = jnp.concatenate(cols, axis=1)


def _sc_gather(qflat, idxflat):
    rows, d = idxflat.shape[0], qflat.shape[1]
    ch = 128
    chunks = rows // (SC_WORKERS * ch)

    mesh = plsc.VectorSubcoreMesh(core_axis_name="c", subcore_axis_name="s")

    @functools.partial(
        pl.kernel,
        out_type=jax.ShapeDtypeStruct((rows, d), qflat.dtype),
        mesh=mesh,
        scratch_types=[
            pltpu.VMEM((ch,), jnp.int32),
            pltpu.VMEM((ch, d), qflat.dtype),
            pltpu.SemaphoreType.DMA,
        ],
    )
    def gk(q_hbm, idx_hbm, out_hbm, idx_v, rows_v, sem):
        c = lax.axis_index("c")
        s = lax.axis_index("s")
        wid = s * SC_CORES + c

        def body(j, carry):
            base = (wid * chunks + j) * ch
            pltpu.sync_copy(idx_hbm.at[pl.ds(base, ch)], idx_v)
            pltpu.async_copy(q_hbm.at[idx_v], rows_v, sem).wait()
            pltpu.sync_copy(rows_v, out_hbm.at[pl.ds(base, ch)])
            return carry

        lax.fori_loop(0, chunks, body, 0)

    return gk(qflat, idxflat)


def _stats1_body(g_ref, q_ref, b1_ref, o_ref):
    t = pl.program_id(0)
    tn = q_ref.shape[0]
    d = q_ref.shape[1]
    g = g_ref[...]
    qm = q_ref[...] - b1_ref[...]
    qb = jnp.broadcast_to(qm[:, None, :],
                          (tn, KNN, d)).reshape(tn * KNN, d)
    h1 = g - qb
    s = jnp.sum(h1, axis=0, keepdims=True)
    ss = jnp.sum(h1 * h1, axis=0, keepdims=True)
    upd = jnp.concatenate([s, ss], axis=0)

    @pl.when(t == 0)
    def _():
        o_ref[...] = jnp.zeros_like(o_ref)

    o_ref[...] += upd


def _main_body(nrows, g_ref, q_ref, st1_ref, b1_ref, g1_ref, be1_ref, w2_ref,
               b2_ref, mmax_ref, mmin_ref, st2_ref):
    t = pl.program_id(0)
    tn = q_ref.shape[0]
    d = q_ref.shape[1]
    st = st1_ref[...]
    mean1 = st[0:1, :] / nrows
    var1 = st[1:2, :] / nrows - mean1 * mean1
    sc1 = g1_ref[...] * lax.rsqrt(var1 + EPS)
    c1 = be1_ref[...] - mean1 * sc1

    g = g_ref[...]
    off = (q_ref[...] - b1_ref[...]) * sc1 - c1
    offb = jnp.broadcast_to(off[:, None, :],
                            (tn, KNN, d)).reshape(tn * KNN, d)
    a = jnp.maximum(g * sc1 - offb, 0.0).astype(jnp.bfloat16)
    h2 = jnp.dot(a, w2_ref[...],
                 preferred_element_type=jnp.float32) + b2_ref[...]

    s = jnp.sum(h2, axis=0, keepdims=True)
    ss = jnp.sum(h2 * h2, axis=0, keepdims=True)

    @pl.when(t == 0)
    def _():
        st2_ref[...] = jnp.zeros_like(st2_ref)

    st2_ref[...] += jnp.concatenate([s, ss], axis=0)

    h3 = h2.reshape(tn, KNN, d)
    mmax_ref[...] = jnp.max(h3, axis=1)
    mmin_ref[...] = jnp.min(h3, axis=1)


def _final_body(nrows, mmax_ref, mmin_ref, st2_ref, g2_ref, be2_ref, o_ref):
    st = st2_ref[...]
    mean2 = st[0:1, :] / nrows
    var2 = st[1:2, :] / nrows - mean2 * mean2
    g2 = g2_ref[...]
    sc2 = g2 * lax.rsqrt(var2 + EPS)
    c2 = be2_ref[...] - mean2 * sc2
    pick = jnp.where(g2 >= 0.0, mmax_ref[...], mmin_ref[...])
    o_ref[...] = jnp.maximum(pick * sc2 + c2, 0.0)


def kernel(xyz, features, W1, b1, g1, be1, W2, b2, g2, be2):
    B, N, _ = xyz.shape
    D = W1.shape[1]
    R = B * N * KNN
    BN = B * N
    RB = N * KNN

    x_cat = jnp.concatenate([xyz, features], axis=2).reshape(BN, D)
    b1r = b1.reshape(1, D)
    g1r = g1.reshape(1, D)
    be1r = be1.reshape(1, D)
    b2r = b2.reshape(1, D)
    g2r = g2.reshape(1, D)
    be2r = be2.reshape(1, D)
    w2bf = W2.astype(jnp.bfloat16)

    TQ = 1024
    q = pl.pallas_call(
        _q_body,
        grid=(BN // TQ,),
        in_specs=[
            pl.BlockSpec((TQ, D), lambda t: (t, 0)),
            pl.BlockSpec((D, D), lambda t: (0, 0)),
        ],
        out_specs=pl.BlockSpec((TQ, D), lambda t: (t, 0)),
        out_shape=jax.ShapeDtypeStruct((BN, D), jnp.float32),
    )(x_cat, W1)
    q = q.reshape(B, N, D)

    TN = 256
    xyzt = xyz.transpose(0, 2, 1)
    topk_call = pl.pallas_call(
        _topk_body,
        grid=(N // TN,),
        in_specs=[
            pl.BlockSpec((TN, 3), lambda t: (t, 0)),
            pl.BlockSpec((3, N), lambda t: (0, 0)),
        ],
        out_specs=pl.BlockSpec((TN, KNN), lambda t: (t, 0)),
        out_shape=jax.ShapeDtypeStruct((N, KNN), jnp.int32),
    )
    g_rows = []
    for b in range(B):
        idx_b = topk_call(xyz[b], xyzt[b])
        g_rows.append(_sc_gather(q[b], idx_b.reshape(RB)))

    TS = 256
    stats_call = pl.pallas_call(
        _stats1_body,
        grid=(N // TS,),
        in_specs=[
            pl.BlockSpec((TS * KNN, D), lambda t: (t, 0)),
            pl.BlockSpec((TS, D), lambda t: (t, 0)),
            pl.BlockSpec((1, D), lambda t: (0, 0)),
        ],
        out_specs=pl.BlockSpec((2, D), lambda t: (0, 0)),
        out_shape=jax.ShapeDtypeStruct((2, D), jnp.float32),
    )
    stats1 = sum(stats_call(g_rows[b], q[b], b1r) for b in range(B))

    TM = 256
    main_call = pl.pallas_call(
        functools.partial(_main_body, float(R)),
        grid=(N // TM,),
        in_specs=[
            pl.BlockSpec((TM * KNN, D), lambda t: (t, 0)),
            pl.BlockSpec((TM, D), lambda t: (t, 0)),
            pl.BlockSpec((2, D), lambda t: (0, 0)),
            pl.BlockSpec((1, D), lambda t: (0, 0)),
            pl.BlockSpec((1, D), lambda t: (0, 0)),
            pl.BlockSpec((1, D), lambda t: (0, 0)),
            pl.BlockSpec((D, D), lambda t: (0, 0)),
            pl.BlockSpec((1, D), lambda t: (0, 0)),
        ],
        out_specs=[
            pl.BlockSpec((TM, D), lambda t: (t, 0)),
            pl.BlockSpec((TM, D), lambda t: (t, 0)),
            pl.BlockSpec((2, D), lambda t: (0, 0)),
        ],
        out_shape=[
            jax.ShapeDtypeStruct((N, D), jnp.float32),
            jax.ShapeDtypeStruct((N, D), jnp.float32),
            jax.ShapeDtypeStruct((2, D), jnp.float32),
        ],
    )
    mmax, mmin, st2 = [], [], []
    for b in range(B):
        a_, b_, c_ = main_call(g_rows[b], q[b], stats1, b1r, g1r, be1r,
                               w2bf, b2r)
        mmax.append(a_)
        mmin.append(b_)
        st2.append(c_)
    stats2 = sum(st2)

    TF = 512
    final_call = pl.pallas_call(
        functools.partial(_final_body, float(R)),
        grid=(N // TF,),
        in_specs=[
            pl.BlockSpec((TF, D), lambda t: (t, 0)),
            pl.BlockSpec((TF, D), lambda t: (t, 0)),
            pl.BlockSpec((2, D), lambda t: (0, 0)),
            pl.BlockSpec((1, D), lambda t: (0, 0)),
            pl.BlockSpec((1, D), lambda t: (0, 0)),
        ],
        out_specs=pl.BlockSpec((TF, D), lambda t: (t, 0)),
        out_shape=jax.ShapeDtypeStruct((N, D), jnp.float32),
    )
    out = [final_call(mmax[b], mmin[b], stats2, g2r, be2r) for b in range(B)]
    return jnp.stack(out, axis=0)


# speedup vs baseline: 1.3613x; 1.1023x over previous
import functools

import jax
import jax.numpy as jnp
from jax import lax
from jax.experimental import pallas as pl
from jax.experimental.pallas import tpu as pltpu
from jax.experimental.pallas import tpu_sc as plsc

KNN = 16
EPS = 1e-5

SC_CORES = 2
SC_SUBCORES = 16
SC_WORKERS = SC_CORES * SC_SUBCORES


def _q_body(x_ref, w_ref, o_ref):
    o_ref[...] = jnp.dot(x_ref[...], w_ref[...],
                         preferred_element_type=jnp.float32)


_SORT16 = [
    (0, 1), (2, 3), (4, 5), (6, 7), (8, 9), (10, 11), (12, 13), (14, 15),
    (0, 2), (1, 3), (4, 6), (5, 7), (8, 10), (9, 11), (12, 14), (13, 15),
    (1, 2), (5, 6), (9, 10), (13, 14), (0, 4), (1, 5), (2, 6), (3, 7),
    (8, 12), (9, 13), (10, 14), (11, 15), (2, 4), (3, 5), (10, 12), (11, 13),
    (1, 2), (3, 4), (5, 6), (9, 10), (11, 12), (13, 14), (0, 8), (1, 9),
    (2, 10), (3, 11), (4, 12), (5, 13), (6, 14), (7, 15), (4, 8), (5, 9),
    (6, 10), (7, 11), (2, 4), (3, 5), (6, 8), (7, 9), (10, 12), (11, 13),
    (1, 2), (3, 4), (5, 6), (7, 8), (9, 10), (11, 12), (13, 14),
]


def _topk_body(xyz_ref, xyzt_ref, o_ref):
    x = xyz_ref[...]
    xt = xyzt_ref[...]
    n = xt.shape[1]
    tn = x.shape[0]
    nlane = 128
    ng = n // nlane
    sq_t = jnp.sum(x * x, axis=1, keepdims=True)
    sq_f = jnp.sum(xt * xt, axis=0, keepdims=True)
    dot = jnp.dot(x, xt, preferred_element_type=jnp.float32)
    d2 = jnp.maximum(sq_t + sq_f - 2.0 * dot, 0.0)

    planes = [d2[:, g * nlane:(g + 1) * nlane] for g in range(ng)]
    lane_iota = lax.broadcasted_iota(jnp.int32, (tn, nlane), 1)
    idxs = [lane_iota + (g * nlane) for g in range(ng)]
    for (a, b) in _SORT16:
        va, vb = planes[a], planes[b]
        ia, ib = idxs[a], idxs[b]
        swap = (vb < va) | ((vb == va) & (ib < ia))
        planes[a] = jnp.where(swap, vb, va)
        planes[b] = jnp.where(swap, va, vb)
        idxs[a] = jnp.where(swap, ib, ia)
        idxs[b] = jnp.where(swap, ia, ib)
    cols = []
    for r in range(KNN):
        m = jnp.min(planes[0], axis=1, keepdims=True)
        cand = jnp.where(planes[0] == m, idxs[0], n)
        i = jnp.min(cand, axis=1, keepdims=True)
        cols.append(i)
        win = cand == i
        for j in range(ng - 1 - r):
            planes[j] = jnp.where(win, planes[j + 1], planes[j])
            idxs[j] = jnp.where(win, idxs[j + 1], idxs[j])
    o_ref[...] = jnp.concatenate(cols, axis=1)


def _sc_gather(qflat, idxflat):
    rows, d = idxflat.shape[0], qflat.shape[1]
    ch = 128
    chunks = rows // (SC_WORKERS * ch)

    mesh = plsc.VectorSubcoreMesh(core_axis_name="c", subcore_axis_name="s")

    @functools.partial(
        pl.kernel,
        out_type=jax.ShapeDtypeStruct((rows, d), qflat.dtype),
        mesh=mesh,
        scratch_types=[
            pltpu.VMEM((ch,), jnp.int32),
            pltpu.VMEM((ch, d), qflat.dtype),
            pltpu.SemaphoreType.DMA,
        ],
    )
    def gk(q_hbm, idx_hbm, out_hbm, idx_v, rows_v, sem):
        c = lax.axis_index("c")
        s = lax.axis_index("s")
        wid = s * SC_CORES + c

        def body(j, carry):
            base = (wid * chunks + j) * ch
            pltpu.sync_copy(idx_hbm.at[pl.ds(base, ch)], idx_v)
            pltpu.async_copy(q_hbm.at[idx_v], rows_v, sem).wait()
            pltpu.sync_copy(rows_v, out_hbm.at[pl.ds(base, ch)])
            return carry

        lax.fori_loop(0, chunks, body, 0)

    return gk(qflat, idxflat)


def _stats1_body(g_ref, q_ref, b1_ref, o_ref):
    t = pl.program_id(0)
    tn = q_ref.shape[0]
    d = q_ref.shape[1]
    g = g_ref[...]
    qm = q_ref[...] - b1_ref[...]
    qb = jnp.broadcast_to(qm[:, None, :],
                          (tn, KNN, d)).reshape(tn * KNN, d)
    h1 = g - qb
    s = jnp.sum(h1, axis=0, keepdims=True)
    ss = jnp.sum(h1 * h1, axis=0, keepdims=True)
    upd = jnp.concatenate([s, ss], axis=0)

    @pl.when(t == 0)
    def _():
        o_ref[...] = jnp.zeros_like(o_ref)

    o_ref[...] += upd


def _main_body(nrows, g_ref, q_ref, st1_ref, b1_ref, g1_ref, be1_ref, w2_ref,
               b2_ref, mmax_ref, mmin_ref, st2_ref):
    t = pl.program_id(0)
    tn = q_ref.shape[0]
    d = q_ref.shape[1]
    st = st1_ref[...]
    mean1 = st[0:1, :] / nrows
    var1 = st[1:2, :] / nrows - mean1 * mean1
    sc1 = g1_ref[...] * lax.rsqrt(var1 + EPS)
    c1 = be1_ref[...] - mean1 * sc1

    g = g_ref[...]
    off = (q_ref[...] - b1_ref[...]) * sc1 - c1
    offb = jnp.broadcast_to(off[:, None, :],
                            (tn, KNN, d)).reshape(tn * KNN, d)
    a = jnp.maximum(g * sc1 - offb, 0.0).astype(jnp.bfloat16)
    h2 = jnp.dot(a, w2_ref[...],
                 preferred_element_type=jnp.float32) + b2_ref[...]

    s = jnp.sum(h2, axis=0, keepdims=True)
    ss = jnp.sum(h2 * h2, axis=0, keepdims=True)

    @pl.when(t == 0)
    def _():
        st2_ref[...] = jnp.zeros_like(st2_ref)

    st2_ref[...] += jnp.concatenate([s, ss], axis=0)

    h3 = h2.reshape(tn, KNN, d)
    mmax_ref[...] = jnp.max(h3, axis=1)
    mmin_ref[...] = jnp.min(h3, axis=1)


def _final_body(nrows, mmax_ref, mmin_ref, st2_ref, g2_ref, be2_ref, o_ref):
    st = st2_ref[...]
    mean2 = st[0:1, :] / nrows
    var2 = st[1:2, :] / nrows - mean2 * mean2
    g2 = g2_ref[...]
    sc2 = g2 * lax.rsqrt(var2 + EPS)
    c2 = be2_ref[...] - mean2 * sc2
    pick = jnp.where(g2 >= 0.0, mmax_ref[...], mmin_ref[...])
    o_ref[...] = jnp.maximum(pick * sc2 + c2, 0.0)


def kernel(xyz, features, W1, b1, g1, be1, W2, b2, g2, be2):
    B, N, _ = xyz.shape
    D = W1.shape[1]
    R = B * N * KNN
    BN = B * N
    RB = N * KNN

    x_cat = jnp.concatenate([xyz, features], axis=2).reshape(BN, D)
    b1r = b1.reshape(1, D)
    g1r = g1.reshape(1, D)
    be1r = be1.reshape(1, D)
    b2r = b2.reshape(1, D)
    g2r = g2.reshape(1, D)
    be2r = be2.reshape(1, D)
    w2bf = W2.astype(jnp.bfloat16)

    TQ = 1024
    q = pl.pallas_call(
        _q_body,
        grid=(BN // TQ,),
        in_specs=[
            pl.BlockSpec((TQ, D), lambda t: (t, 0)),
            pl.BlockSpec((D, D), lambda t: (0, 0)),
        ],
        out_specs=pl.BlockSpec((TQ, D), lambda t: (t, 0)),
        out_shape=jax.ShapeDtypeStruct((BN, D), jnp.float32),
    )(x_cat, W1)
    q = q.reshape(B, N, D)

    TN = 256
    xyzt = xyz.transpose(0, 2, 1)
    topk_call = pl.pallas_call(
        _topk_body,
        grid=(N // TN,),
        in_specs=[
            pl.BlockSpec((TN, 3), lambda t: (t, 0)),
            pl.BlockSpec((3, N), lambda t: (0, 0)),
        ],
        out_specs=pl.BlockSpec((TN, KNN), lambda t: (t, 0)),
        out_shape=jax.ShapeDtypeStruct((N, KNN), jnp.int32),
    )
    g_rows = []
    for b in range(B):
        idx_b = topk_call(xyz[b], xyzt[b])
        g_rows.append(_sc_gather(q[b], idx_b.reshape(RB)))

    TS = 256
    stats_call = pl.pallas_call(
        _stats1_body,
        grid=(N // TS,),
        in_specs=[
            pl.BlockSpec((TS * KNN, D), lambda t: (t, 0)),
            pl.BlockSpec((TS, D), lambda t: (t, 0)),
            pl.BlockSpec((1, D), lambda t: (0, 0)),
        ],
        out_specs=pl.BlockSpec((2, D), lambda t: (0, 0)),
        out_shape=jax.ShapeDtypeStruct((2, D), jnp.float32),
    )
    stats1 = sum(stats_call(g_rows[b], q[b], b1r) for b in range(B))

    TM = 256
    main_call = pl.pallas_call(
        functools.partial(_main_body, float(R)),
        grid=(N // TM,),
        in_specs=[
            pl.BlockSpec((TM * KNN, D), lambda t: (t, 0)),
            pl.BlockSpec((TM, D), lambda t: (t, 0)),
            pl.BlockSpec((2, D), lambda t: (0, 0)),
            pl.BlockSpec((1, D), lambda t: (0, 0)),
            pl.BlockSpec((1, D), lambda t: (0, 0)),
            pl.BlockSpec((1, D), lambda t: (0, 0)),
            pl.BlockSpec((D, D), lambda t: (0, 0)),
            pl.BlockSpec((1, D), lambda t: (0, 0)),
        ],
        out_specs=[
            pl.BlockSpec((TM, D), lambda t: (t, 0)),
            pl.BlockSpec((TM, D), lambda t: (t, 0)),
            pl.BlockSpec((2, D), lambda t: (0, 0)),
        ],
        out_shape=[
            jax.ShapeDtypeStruct((N, D), jnp.float32),
            jax.ShapeDtypeStruct((N, D), jnp.float32),
            jax.ShapeDtypeStruct((2, D), jnp.float32),
        ],
    )
    mmax, mmin, st2 = [], [], []
    for b in range(B):
        a_, b_, c_ = main_call(g_rows[b], q[b], stats1, b1r, g1r, be1r,
                               w2bf, b2r)
        mmax.append(a_)
        mmin.append(b_)
        st2.append(c_)
    stats2 = sum(st2)

    TF = 512
    final_call = pl.pallas_call(
        functools.partial(_final_body, float(R)),
        grid=(N // TF,),
        in_specs=[
            pl.BlockSpec((TF, D), lambda t: (t, 0)),
            pl.BlockSpec((TF, D), lambda t: (t, 0)),
            pl.BlockSpec((2, D), lambda t: (0, 0)),
            pl.BlockSpec((1, D), lambda t: (0, 0)),
            pl.BlockSpec((1, D), lambda t: (0, 0)),
        ],
        out_specs=pl.BlockSpec((TF, D), lambda t: (t, 0)),
        out_shape=jax.ShapeDtypeStruct((N, D), jnp.float32),
    )
    out = [final_call(mmax[b], mmin[b], stats2, g2r, be2r) for b in range(B)]
    return jnp.stack(out, axis=0)


# speedup vs baseline: 1.3970x; 1.0263x over previous
import functools

import jax
import jax.numpy as jnp
from jax import lax
from jax.experimental import pallas as pl
from jax.experimental.pallas import tpu as pltpu
from jax.experimental.pallas import tpu_sc as plsc

KNN = 16
EPS = 1e-5

SC_CORES = 2
SC_SUBCORES = 16
SC_WORKERS = SC_CORES * SC_SUBCORES


def _q_body(x_ref, w_ref, o_ref):
    o_ref[...] = jnp.dot(x_ref[...], w_ref[...],
                         preferred_element_type=jnp.float32)


_SORT16 = [
    (0, 1), (2, 3), (4, 5), (6, 7), (8, 9), (10, 11), (12, 13), (14, 15),
    (0, 2), (1, 3), (4, 6), (5, 7), (8, 10), (9, 11), (12, 14), (13, 15),
    (1, 2), (5, 6), (9, 10), (13, 14), (0, 4), (1, 5), (2, 6), (3, 7),
    (8, 12), (9, 13), (10, 14), (11, 15), (2, 4), (3, 5), (10, 12), (11, 13),
    (1, 2), (3, 4), (5, 6), (9, 10), (11, 12), (13, 14), (0, 8), (1, 9),
    (2, 10), (3, 11), (4, 12), (5, 13), (6, 14), (7, 15), (4, 8), (5, 9),
    (6, 10), (7, 11), (2, 4), (3, 5), (6, 8), (7, 9), (10, 12), (11, 13),
    (1, 2), (3, 4), (5, 6), (7, 8), (9, 10), (11, 12), (13, 14),
]


def _topk_body(xyz_ref, xyzt_ref, o_ref):
    x = xyz_ref[...]
    xt = xyzt_ref[...]
    n = xt.shape[1]
    tn = x.shape[0]
    nlane = 128
    ng = n // nlane
    sq_t = jnp.sum(x * x, axis=1, keepdims=True)
    sq_f = jnp.sum(xt * xt, axis=0, keepdims=True)
    dot = jnp.dot(x, xt, preferred_element_type=jnp.float32)
    d2 = jnp.maximum(sq_t + sq_f - 2.0 * dot, 0.0)

    planes = [d2[:, g * nlane:(g + 1) * nlane] for g in range(ng)]
    lane_iota = lax.broadcasted_iota(jnp.int32, (tn, nlane), 1)
    idxs = [lane_iota + (g * nlane) for g in range(ng)]
    for (a, b) in _SORT16:
        va, vb = planes[a], planes[b]
        ia, ib = idxs[a], idxs[b]
        swap = (vb < va) | ((vb == va) & (ib < ia))
        planes[a] = jnp.where(swap, vb, va)
        planes[b] = jnp.where(swap, va, vb)
        idxs[a] = jnp.where(swap, ib, ia)
        idxs[b] = jnp.where(swap, ia, ib)
    cols = []
    for r in range(KNN):
        m = jnp.min(planes[0], axis=1, keepdims=True)
        cand = jnp.where(planes[0] == m, idxs[0], n)
        i = jnp.min(cand, axis=1, keepdims=True)
        cols.append(i)
        win = cand == i
        for j in range(ng - 1 - r):
            planes[j] = jnp.where(win, planes[j + 1], planes[j])
            idxs[j] = jnp.where(win, idxs[j + 1], idxs[j])
    o_ref[...] = jnp.concatenate(cols, axis=1)


def _sc_gather(qflat, idxflat):
    rows, d = idxflat.shape[0], qflat.shape[1]
    ch = 128
    chunks = rows // (SC_WORKERS * ch)

    mesh = plsc.VectorSubcoreMesh(core_axis_name="c", subcore_axis_name="s")

    @functools.partial(
        pl.kernel,
        out_type=jax.ShapeDtypeStruct((rows, d), qflat.dtype),
        mesh=mesh,
        scratch_types=[
            pltpu.VMEM((ch,), jnp.int32),
            pltpu.VMEM((ch, d), qflat.dtype),
            pltpu.SemaphoreType.DMA,
        ],
    )
    def gk(q_hbm, idx_hbm, out_hbm, idx_v, rows_v, sem):
        c = lax.axis_index("c")
        s = lax.axis_index("s")
        wid = s * SC_CORES + c

        def body(j, carry):
            base = (wid * chunks + j) * ch
            pltpu.sync_copy(idx_hbm.at[pl.ds(base, ch)], idx_v)
            pltpu.async_copy(q_hbm.at[idx_v], rows_v, sem).wait()
            pltpu.sync_copy(rows_v, out_hbm.at[pl.ds(base, ch)])
            return carry

        lax.fori_loop(0, chunks, body, 0)

    return gk(qflat, idxflat)


def _stats1_body(g_ref, q_ref, b1_ref, o_ref):
    t = pl.program_id(0)
    tn = q_ref.shape[0]
    d = q_ref.shape[1]
    g = g_ref[...]
    qm = q_ref[...] - b1_ref[...]
    qb = jnp.broadcast_to(qm[:, None, :],
                          (tn, KNN, d)).reshape(tn * KNN, d)
    h1 = g - qb
    s = jnp.sum(h1, axis=0, keepdims=True)
    ss = jnp.sum(h1 * h1, axis=0, keepdims=True)
    upd = jnp.concatenate([s, ss], axis=0)

    @pl.when(t == 0)
    def _():
        o_ref[...] = jnp.zeros_like(o_ref)

    o_ref[...] += upd


def _main_body(nrows, g_ref, q_ref, st1_ref, b1_ref, g1_ref, be1_ref, w2_ref,
               b2_ref, mmax_ref, st2_ref):
    t = pl.program_id(0)
    tn = q_ref.shape[0]
    d = q_ref.shape[1]
    st = st1_ref[...]
    mean1 = st[0:1, :] / nrows
    var1 = st[1:2, :] / nrows - mean1 * mean1
    sc1 = g1_ref[...] * lax.rsqrt(var1 + EPS)
    c1 = be1_ref[...] - mean1 * sc1

    g = g_ref[...]
    off = (q_ref[...] - b1_ref[...]) * sc1 - c1
    offb = jnp.broadcast_to(off[:, None, :],
                            (tn, KNN, d)).reshape(tn * KNN, d)
    a = jnp.maximum(g * sc1 - offb, 0.0).astype(jnp.bfloat16)
    h2 = jnp.dot(a, w2_ref[...],
                 preferred_element_type=jnp.float32) + b2_ref[...]

    s = jnp.sum(h2, axis=0, keepdims=True)
    ss = jnp.sum(h2 * h2, axis=0, keepdims=True)

    @pl.when(t == 0)
    def _():
        st2_ref[...] = jnp.zeros_like(st2_ref)

    st2_ref[...] += jnp.concatenate([s, ss], axis=0)

    h3 = h2.reshape(tn, KNN, d)
    mmax_ref[...] = jnp.max(h3, axis=1)


def _final_body(nrows, mmax_ref, st2_ref, g2_ref, be2_ref, o_ref):
    st = st2_ref[...]
    mean2 = st[0:1, :] / nrows
    var2 = st[1:2, :] / nrows - mean2 * mean2
    sc2 = jnp.abs(g2_ref[...]) * lax.rsqrt(var2 + EPS)
    o_ref[...] = jnp.maximum((mmax_ref[...] - mean2) * sc2 + be2_ref[...],
                             0.0)


def kernel(xyz, features, W1, b1, g1, be1, W2, b2, g2, be2):
    B, N, _ = xyz.shape
    D = W1.shape[1]
    R = B * N * KNN
    BN = B * N
    RB = N * KNN

    x_cat = jnp.concatenate([xyz, features], axis=2).reshape(BN, D)
    b1r = b1.reshape(1, D)
    g1r = g1.reshape(1, D)
    be1r = be1.reshape(1, D)
    b2r = b2.reshape(1, D)
    g2r = g2.reshape(1, D)
    be2r = be2.reshape(1, D)
    s2 = jnp.sign(g2)
    w2bf = (W2 * s2[None, :]).astype(jnp.bfloat16)
    b2r = b2r * s2.reshape(1, D)

    TQ = 1024
    q = pl.pallas_call(
        _q_body,
        grid=(BN // TQ,),
        in_specs=[
            pl.BlockSpec((TQ, D), lambda t: (t, 0)),
            pl.BlockSpec((D, D), lambda t: (0, 0)),
        ],
        out_specs=pl.BlockSpec((TQ, D), lambda t: (t, 0)),
        out_shape=jax.ShapeDtypeStruct((BN, D), jnp.float32),
    )(x_cat, W1)
    q = q.reshape(B, N, D)

    TN = 256
    xyzt = xyz.transpose(0, 2, 1)
    topk_call = pl.pallas_call(
        _topk_body,
        grid=(N // TN,),
        in_specs=[
            pl.BlockSpec((TN, 3), lambda t: (t, 0)),
            pl.BlockSpec((3, N), lambda t: (0, 0)),
        ],
        out_specs=pl.BlockSpec((TN, KNN), lambda t: (t, 0)),
        out_shape=jax.ShapeDtypeStruct((N, KNN), jnp.int32),
    )
    g_rows = []
    for b in range(B):
        idx_b = topk_call(xyz[b], xyzt[b])
        g_rows.append(_sc_gather(q[b], idx_b.reshape(RB)))

    TS = 256
    stats_call = pl.pallas_call(
        _stats1_body,
        grid=(N // TS,),
        in_specs=[
            pl.BlockSpec((TS * KNN, D), lambda t: (t, 0)),
            pl.BlockSpec((TS, D), lambda t: (t, 0)),
            pl.BlockSpec((1, D), lambda t: (0, 0)),
        ],
        out_specs=pl.BlockSpec((2, D), lambda t: (0, 0)),
        out_shape=jax.ShapeDtypeStruct((2, D), jnp.float32),
    )
    stats1 = sum(stats_call(g_rows[b], q[b], b1r) for b in range(B))

    TM = 256
    main_call = pl.pallas_call(
        functools.partial(_main_body, float(R)),
        grid=(N // TM,),
        in_specs=[
            pl.BlockSpec((TM * KNN, D), lambda t: (t, 0)),
            pl.BlockSpec((TM, D), lambda t: (t, 0)),
            pl.BlockSpec((2, D), lambda t: (0, 0)),
            pl.BlockSpec((1, D), lambda t: (0, 0)),
            pl.BlockSpec((1, D), lambda t: (0, 0)),
            pl.BlockSpec((1, D), lambda t: (0, 0)),
            pl.BlockSpec((D, D), lambda t: (0, 0)),
            pl.BlockSpec((1, D), lambda t: (0, 0)),
        ],
        out_specs=[
            pl.BlockSpec((TM, D), lambda t: (t, 0)),
            pl.BlockSpec((2, D), lambda t: (0, 0)),
        ],
        out_shape=[
            jax.ShapeDtypeStruct((N, D), jnp.float32),
            jax.ShapeDtypeStruct((2, D), jnp.float32),
        ],
    )
    mmax, st2 = [], []
    for b in range(B):
        a_, c_ = main_call(g_rows[b], q[b], stats1, b1r, g1r, be1r,
                           w2bf, b2r)
        mmax.append(a_)
        st2.append(c_)
    stats2 = sum(st2)

    TF = 512
    final_call = pl.pallas_call(
        functools.partial(_final_body, float(R)),
        grid=(N // TF,),
        in_specs=[
            pl.BlockSpec((TF, D), lambda t: (t, 0)),
            pl.BlockSpec((2, D), lambda t: (0, 0)),
            pl.BlockSpec((1, D), lambda t: (0, 0)),
            pl.BlockSpec((1, D), lambda t: (0, 0)),
        ],
        out_specs=pl.BlockSpec((TF, D), lambda t: (t, 0)),
        out_shape=jax.ShapeDtypeStruct((N, D), jnp.float32),
    )
    out = [final_call(mmax[b], stats2, g2r, be2r) for b in range(B)]
    return jnp.stack(out, axis=0)
